# baseline (speedup 1.0000x reference)
import functools

import numpy as np

import jax
import jax.numpy as jnp
from jax import lax
from jax.experimental import pallas as pl
from jax.experimental.pallas import tpu as pltpu
from jax.experimental.pallas import tpu_sc as plsc

EMBED = 128
HALF = EMBED // 2
N_NODE = 10000
N_EDGE = 320000

NUM_CORES = 2
NUM_SUBCORES = 16
EDGES_PER_TILE = N_EDGE // NUM_SUBCORES
E_BLK = 80
N_BLKS = EDGES_PER_TILE // E_BLK

ROWS_MAIN = 624
ROWS_LAST = 640

_TC_ROWS = 2000
_TC_GRID = N_NODE // _TC_ROWS


def _split(x):
    return x[:, :HALF], x[:, HALF:]


def _tc_pre_body(f_ref, v_ref, w1_ref, w2_ref, bm_ref, a_ref, b_ref):
    a = jnp.dot(f_ref[...], w1_ref[...],
                preferred_element_type=jnp.float32) + bm_ref[...]
    b = jnp.dot(v_ref[...], w2_ref[...], preferred_element_type=jnp.float32)
    a = a.astype(jnp.bfloat16)
    b = b.astype(jnp.bfloat16)
    a_ref[0], a_ref[1] = _split(a)
    b_ref[0], b_ref[1] = _split(b)


def _tc_mid_body(f_ref, v_ref, p_ref, wc1_ref, wc2_ref, bc_ref,
                 w3_ref, w4_ref, bm2_ref, nf_ref, c_ref, d_ref):
    aggr = jnp.concatenate([p_ref[0], p_ref[1]], axis=-1)
    nf = jax.nn.relu(
        jnp.dot(f_ref[...], wc1_ref[...], preferred_element_type=jnp.float32)
        + jnp.dot(aggr, wc2_ref[...], preferred_element_type=jnp.float32)
        + bc_ref[...])
    nf_ref[...] = nf
    c = jnp.dot(v_ref[...], w3_ref[...], preferred_element_type=jnp.float32)
    d = jnp.dot(nf, w4_ref[...],
                preferred_element_type=jnp.float32) + bm2_ref[...]
    c = c.astype(jnp.bfloat16)
    d = d.astype(jnp.bfloat16)
    c_ref[0], c_ref[1] = _split(c)
    d_ref[0], d_ref[1] = _split(d)


def _tc_post_body(v_ref, q_ref, wc3_ref, wc4_ref, bc2_ref, nv_ref):
    aggr = jnp.concatenate([q_ref[0], q_ref[1]], axis=-1)
    nv_ref[...] = v_ref[...] + jax.nn.relu(
        jnp.dot(v_ref[...], wc3_ref[...], preferred_element_type=jnp.float32)
        + jnp.dot(aggr, wc4_ref[...], preferred_element_type=jnp.float32)
        + bc2_ref[...])


def _row_spec():
    return pl.BlockSpec((_TC_ROWS, EMBED), lambda i: (i, 0))


def _full_spec(shape):
    n = len(shape)
    return pl.BlockSpec(shape, lambda i: (0,) * n)


def _split_spec():
    return pl.BlockSpec((NUM_CORES, _TC_ROWS, HALF), lambda i: (0, i, 0))


_mat = functools.partial(jax.ShapeDtypeStruct, dtype=jnp.float32)
_bmat = functools.partial(jax.ShapeDtypeStruct, dtype=jnp.bfloat16)
_SPLIT_SHAPE = (NUM_CORES, N_NODE, HALF)

_POS = np.arange(HALF)
_SIG_HALF = 32 * (_POS // 32) + np.where(
    _POS % 32 < 16, 2 * (_POS % 32), 2 * (_POS % 32 - 16) + 1)
_SIGMA = np.concatenate([_SIG_HALF, HALF + _SIG_HALF])


def _tc_pre(factors, variables, w1, w2, bm):
    return pl.pallas_call(
        _tc_pre_body,
        grid=(_TC_GRID,),
        in_specs=[_row_spec(), _row_spec(), _full_spec((EMBED, EMBED)),
                  _full_spec((EMBED, EMBED)), _full_spec((1, EMBED))],
        out_specs=[_split_spec(), _split_spec()],
        out_shape=[_bmat(_SPLIT_SHAPE), _bmat(_SPLIT_SHAPE)],
    )(factors, variables, w1, w2, bm)


def _tc_mid(factors, variables, part, wc1, wc2, bc, w3, w4, bm2):
    return pl.pallas_call(
        _tc_mid_body,
        grid=(_TC_GRID,),
        in_specs=[_row_spec(), _row_spec(), _split_spec(),
                  _full_spec((EMBED, EMBED)), _full_spec((EMBED, EMBED)),
                  _full_spec((1, EMBED)), _full_spec((EMBED, EMBED)),
                  _full_spec((EMBED, EMBED)), _full_spec((1, EMBED))],
        out_specs=[_row_spec(), _split_spec(), _split_spec()],
        out_shape=[_mat((N_NODE, EMBED)), _bmat(_SPLIT_SHAPE),
                   _bmat(_SPLIT_SHAPE)],
    )(factors, variables, part, wc1, wc2, bc, w3, w4, bm2)


def _tc_post(variables, part, wc3, wc4, bc2):
    return pl.pallas_call(
        _tc_post_body,
        grid=(_TC_GRID,),
        in_specs=[_row_spec(), _split_spec(), _full_spec((EMBED, EMBED)),
                  _full_spec((EMBED, EMBED)), _full_spec((1, EMBED))],
        out_specs=_row_spec(),
        out_shape=_mat((N_NODE, EMBED)),
    )(variables, part, wc3, wc4, bc2)


def _sc_edge_body(g1_hbm, i1_hbm, g2_hbm, i2_hbm, zero_hbm, out_hbm,
                  i1_all, i2_all, a0, a1, a2, a3, s0, s1, aggr_sh,
                  sga0, sga1, sga2, sga3, sgb0, sgb1, sgb2, sgb3,
                  ssc0, ssc1):
    buf_a, buf_s = (a0, a1, a2, a3), (s0, s1)
    sem_ga, sem_gb = (sga0, sga1, sga2, sga3), (sgb0, sgb1, sgb2, sgb3)
    sem_sc = (ssc0, ssc1)
    c = lax.axis_index("c")
    s = lax.axis_index("s")
    row_off = pl.multiple_of(s * ROWS_MAIN, 8)

    @pl.when(s < NUM_SUBCORES - 1)
    def _zero_main():
        pltpu.sync_copy(zero_hbm.at[pl.ds(0, ROWS_MAIN)],
                        aggr_sh.at[pl.ds(row_off, ROWS_MAIN)])

    @pl.when(s == NUM_SUBCORES - 1)
    def _zero_last():
        pltpu.sync_copy(zero_hbm, aggr_sh.at[pl.ds(row_off, ROWS_LAST)])

    pltpu.sync_copy(i1_hbm.at[s], i1_all)
    pltpu.sync_copy(i2_hbm.at[s], i2_all)
    plsc.subcore_barrier()

    def issue_g1(blk, q):
        pltpu.async_copy(g1_hbm.at[c].at[i1_all.at[blk]], buf_a[q], sem_ga[q])

    def wait_g1(blk, q):
        pltpu.make_async_copy(g1_hbm.at[c].at[i1_all.at[blk]], buf_a[q],
                              sem_ga[q]).wait()

    def issue_g2add(blk, q):
        pltpu.async_copy(g2_hbm.at[c].at[i2_all.at[blk]], buf_a[q],
                         sem_gb[q], add=True)

    def wait_g2add(blk, q):
        pltpu.make_async_copy(g2_hbm.at[c].at[i2_all.at[blk]], buf_a[q],
                              sem_gb[q]).wait()

    def relu_block(q, p):
        a_buf, s_buf = buf_a[q], buf_s[p]

        @pl.loop(0, E_BLK)
        def _relu_row(r):
            for k in range(HALF // 32):
                lo, hi = plsc.unpack(a_buf[r, pl.ds(k * 32, 32)],
                                     format=plsc.PackFormat.INTERLEAVED)
                s_buf[r, pl.ds(k * 32, 16)] = jnp.maximum(lo, 0.0)
                s_buf[r, pl.ds(k * 32 + 16, 16)] = jnp.maximum(hi, 0.0)

    def issue_scatter(blk, p):
        pltpu.async_copy(buf_s[p], aggr_sh.at[i2_all.at[blk]], sem_sc[p],
                         add=True)

    def wait_scatter(blk, p):
        pltpu.make_async_copy(buf_s[p], aggr_sh.at[i2_all.at[blk]],
                              sem_sc[p]).wait()

    def visit(blk, q, p, scatter_wait=True, refill=True, chain=True):
        wait_g2add(blk, q)
        if scatter_wait:
            wait_scatter(blk, p)
        relu_block(q, p)
        if refill:
            issue_g1(blk + 4, q)
        issue_scatter(blk, p)
        if chain:
            cq = (q + 2) % 4
            wait_g1(blk + 2, cq)
            issue_g2add(blk + 2, cq)

    for q in range(4):
        issue_g1(q, q)
    for q in range(2):
        wait_g1(q, q)
        issue_g2add(q, q)
    visit(0, 0, 0, scatter_wait=False)
    visit(1, 1, 1, scatter_wait=False)
    visit(2, 2, 0)
    visit(3, 3, 1)

    @pl.loop(1, (N_BLKS - 8) // 4 + 1)
    def _steady(i):
        for b in range(4):
            visit(i * 4 + b, b, b % 2)

    for blk in range(N_BLKS - 6, N_BLKS):
        visit(blk, blk % 4, blk % 2,
              refill=blk + 4 < N_BLKS, chain=blk + 2 < N_BLKS)

    for p in range(2):
        wait_scatter(N_BLKS - 2 + p, p)

    plsc.subcore_barrier()

    @pl.when(s < NUM_SUBCORES - 1)
    def _out_main():
        rows = pl.ds(row_off, ROWS_MAIN)
        pltpu.sync_copy(aggr_sh.at[rows], out_hbm.at[c, rows])

    @pl.when(s == NUM_SUBCORES - 1)
    def _out_last():
        rows = pl.ds(row_off, ROWS_LAST)
        pltpu.sync_copy(aggr_sh.at[rows], out_hbm.at[c, rows])


@functools.lru_cache(maxsize=None)
def _make_sc_edge_pass():
    return pl.kernel(
        _sc_edge_body,
        out_type=jax.ShapeDtypeStruct(_SPLIT_SHAPE, jnp.float32),
        mesh=plsc.VectorSubcoreMesh(core_axis_name="c",
                                    subcore_axis_name="s"),
        compiler_params=pltpu.CompilerParams(use_tc_tiling_on_sc=False,
                                             needs_layout_passes=False),
        scratch_types=(
            [pltpu.VMEM((N_BLKS, E_BLK), jnp.int32)] * 2
            + [pltpu.VMEM((E_BLK, HALF), jnp.bfloat16)] * 4
            + [pltpu.VMEM((E_BLK, HALF), jnp.float32)] * 2
            + [pltpu.VMEM_SHARED((N_NODE, HALF), jnp.float32)]
            + [pltpu.SemaphoreType.DMA] * 10
        ),
    )


def kernel(variables, factors, edge_index, edge_attr, batch_idx,
           Wm_vf, bm_vf, Wc_vf, bc_vf, Wm_fv, bm_fv, Wc_fv, bc_fv):
    del edge_attr, batch_idx
    src = edge_index[0].astype(jnp.int32).reshape(NUM_SUBCORES, N_BLKS, E_BLK)
    dst = edge_index[1].astype(jnp.int32).reshape(NUM_SUBCORES, N_BLKS, E_BLK)
    zeros = jnp.zeros((ROWS_LAST, HALF), jnp.float32)

    def half(w):
        return w[:EMBED], w[EMBED:]

    w1, w2 = half(Wm_vf)
    wc1, wc2 = half(Wc_vf)
    w3, w4 = half(Wm_fv)
    wc3, wc4 = half(Wc_fv)
    wc2 = wc2[_SIGMA]
    wc4 = wc4[_SIGMA]

    bm = bm_vf.reshape(1, EMBED)
    bc = bc_vf.reshape(1, EMBED)
    bm2 = bm_fv.reshape(1, EMBED)
    bc2 = bc_fv.reshape(1, EMBED)

    sc_edge_pass = _make_sc_edge_pass()

    a_proj, b_proj = _tc_pre(factors, variables, w1, w2, bm)
    aggr1 = sc_edge_pass(b_proj, src, a_proj, dst, zeros)
    new_factors, c_proj, d_proj = _tc_mid(
        factors, variables, aggr1, wc1, wc2, bc, w3, w4, bm2)

    aggr2 = sc_edge_pass(d_proj, dst, c_proj, src, zeros)
    new_variables = _tc_post(variables, aggr2, wc3, wc4, bc2)

    return new_variables, new_factors


# speedup vs baseline: 1.3320x; 1.3320x over previous
import functools

import jax
import jax.numpy as jnp
from jax import lax
from jax.experimental import pallas as pl
from jax.experimental.pallas import tpu as pltpu
from jax.experimental.pallas import tpu_sc as plsc

EMBED = 128
HALF = EMBED // 2
N_NODE = 10000
N_EDGE = 320000

NUM_CORES = 2
NUM_SUBCORES = 16
EDGES_PER_TILE = N_EDGE // NUM_SUBCORES
E_BLK = 80
N_BLKS = EDGES_PER_TILE // E_BLK

ROWS_MAIN = 624
ROWS_LAST = 640

_TC_ROWS = 2000
_TC_GRID = N_NODE // _TC_ROWS


def _split(x):
    return x[:, :HALF], x[:, HALF:]


def _tc_pre_body(f_ref, v_ref, w1_ref, w2_ref, bm_ref, a_ref, b_ref):
    a = jnp.dot(f_ref[...], w1_ref[...],
                preferred_element_type=jnp.float32) + bm_ref[...]
    b = jnp.dot(v_ref[...], w2_ref[...], preferred_element_type=jnp.float32)
    a_ref[0], a_ref[1] = _split(a)
    b_ref[0], b_ref[1] = _split(b)


def _tc_mid_body(f_ref, v_ref, p_ref, wc1_ref, wc2_ref, bc_ref,
                 w3_ref, w4_ref, bm2_ref, nf_ref, c_ref, d_ref):
    aggr = jnp.concatenate([p_ref[0], p_ref[1]], axis=-1)
    nf = jax.nn.relu(
        jnp.dot(f_ref[...], wc1_ref[...], preferred_element_type=jnp.float32)
        + jnp.dot(aggr, wc2_ref[...], preferred_element_type=jnp.float32)
        + bc_ref[...])
    nf_ref[...] = nf
    c = jnp.dot(v_ref[...], w3_ref[...], preferred_element_type=jnp.float32)
    d = jnp.dot(nf, w4_ref[...],
                preferred_element_type=jnp.float32) + bm2_ref[...]
    c_ref[0], c_ref[1] = _split(c)
    d_ref[0], d_ref[1] = _split(d)


def _tc_post_body(v_ref, q_ref, wc3_ref, wc4_ref, bc2_ref, nv_ref):
    aggr = jnp.concatenate([q_ref[0], q_ref[1]], axis=-1)
    nv_ref[...] = v_ref[...] + jax.nn.relu(
        jnp.dot(v_ref[...], wc3_ref[...], preferred_element_type=jnp.float32)
        + jnp.dot(aggr, wc4_ref[...], preferred_element_type=jnp.float32)
        + bc2_ref[...])


def _row_spec():
    return pl.BlockSpec((_TC_ROWS, EMBED), lambda i: (i, 0))


def _full_spec(shape):
    n = len(shape)
    return pl.BlockSpec(shape, lambda i: (0,) * n)


def _split_spec():
    return pl.BlockSpec((NUM_CORES, _TC_ROWS, HALF), lambda i: (0, i, 0))


_mat = functools.partial(jax.ShapeDtypeStruct, dtype=jnp.float32)
_SPLIT_SHAPE = (NUM_CORES, N_NODE, HALF)


def _tc_pre(factors, variables, w1, w2, bm):
    return pl.pallas_call(
        _tc_pre_body,
        grid=(_TC_GRID,),
        in_specs=[_row_spec(), _row_spec(), _full_spec((EMBED, EMBED)),
                  _full_spec((EMBED, EMBED)), _full_spec((1, EMBED))],
        out_specs=[_split_spec(), _split_spec()],
        out_shape=[_mat(_SPLIT_SHAPE), _mat(_SPLIT_SHAPE)],
    )(factors, variables, w1, w2, bm)


def _tc_mid(factors, variables, part, wc1, wc2, bc, w3, w4, bm2):
    return pl.pallas_call(
        _tc_mid_body,
        grid=(_TC_GRID,),
        in_specs=[_row_spec(), _row_spec(), _split_spec(),
                  _full_spec((EMBED, EMBED)), _full_spec((EMBED, EMBED)),
                  _full_spec((1, EMBED)), _full_spec((EMBED, EMBED)),
                  _full_spec((EMBED, EMBED)), _full_spec((1, EMBED))],
        out_specs=[_row_spec(), _split_spec(), _split_spec()],
        out_shape=[_mat((N_NODE, EMBED)), _mat(_SPLIT_SHAPE),
                   _mat(_SPLIT_SHAPE)],
    )(factors, variables, part, wc1, wc2, bc, w3, w4, bm2)


def _tc_post(variables, part, wc3, wc4, bc2):
    return pl.pallas_call(
        _tc_post_body,
        grid=(_TC_GRID,),
        in_specs=[_row_spec(), _split_spec(), _full_spec((EMBED, EMBED)),
                  _full_spec((EMBED, EMBED)), _full_spec((1, EMBED))],
        out_specs=_row_spec(),
        out_shape=_mat((N_NODE, EMBED)),
    )(variables, part, wc3, wc4, bc2)


def _sc_edge_body(swap, g1_hbm, g2_hbm, eidx_hbm, zero_hbm, out_hbm,
                  i1_all, i2_all, a0, a1, a2, a3, s0, s1, aggr_sh,
                  sga0, sga1, sga2, sga3, sgb0, sgb1, sgb2, sgb3,
                  ssc0, ssc1):
    row1 = 1 if swap else 0
    row2 = 1 - row1
    buf_a, buf_s = (a0, a1, a2, a3), (s0, s1)
    sem_ga, sem_gb = (sga0, sga1, sga2, sga3), (sgb0, sgb1, sgb2, sgb3)
    sem_sc = (ssc0, ssc1)
    c = lax.axis_index("c")
    s = lax.axis_index("s")
    row_off = pl.multiple_of(s * ROWS_MAIN, 8)

    @pl.when(s < NUM_SUBCORES - 1)
    def _zero_main():
        pltpu.sync_copy(zero_hbm.at[pl.ds(0, ROWS_MAIN)],
                        aggr_sh.at[pl.ds(row_off, ROWS_MAIN)])

    @pl.when(s == NUM_SUBCORES - 1)
    def _zero_last():
        pltpu.sync_copy(zero_hbm, aggr_sh.at[pl.ds(row_off, ROWS_LAST)])

    edge_base = s * EDGES_PER_TILE
    pltpu.sync_copy(eidx_hbm.at[row1, pl.ds(edge_base, EDGES_PER_TILE)],
                    i1_all)
    pltpu.sync_copy(eidx_hbm.at[row2, pl.ds(edge_base, EDGES_PER_TILE)],
                    i2_all)
    plsc.subcore_barrier()

    def idx1(blk):
        return i1_all.at[pl.ds(blk * E_BLK, E_BLK)]

    def idx2(blk):
        return i2_all.at[pl.ds(blk * E_BLK, E_BLK)]

    def issue_g1(blk, q):
        pltpu.async_copy(g1_hbm.at[c].at[idx1(blk)], buf_a[q], sem_ga[q])

    def wait_g1(blk, q):
        pltpu.make_async_copy(g1_hbm.at[c].at[idx1(blk)], buf_a[q],
                              sem_ga[q]).wait()

    def issue_g2add(blk, q):
        pltpu.async_copy(g2_hbm.at[c].at[idx2(blk)], buf_a[q],
                         sem_gb[q], add=True)

    def wait_g2add(blk, q):
        pltpu.make_async_copy(g2_hbm.at[c].at[idx2(blk)], buf_a[q],
                              sem_gb[q]).wait()

    def relu_block(q, p):
        a_buf, s_buf = buf_a[q], buf_s[p]

        @pl.loop(0, E_BLK)
        def _relu_row(r):
            for cb in range(HALF // 16):
                sl = pl.ds(cb * 16, 16)
                s_buf[r, sl] = jnp.maximum(a_buf[r, sl], 0.0)

    def issue_scatter(blk, p):
        pltpu.async_copy(buf_s[p], aggr_sh.at[idx2(blk)], sem_sc[p],
                         add=True)

    def wait_scatter(blk, p):
        pltpu.make_async_copy(buf_s[p], aggr_sh.at[idx2(blk)],
                              sem_sc[p]).wait()

    def visit(blk, q, p, scatter_wait=True, refill=True, chain=True):
        wait_g2add(blk, q)
        if scatter_wait:
            wait_scatter(blk, p)
        relu_block(q, p)
        if refill:
            issue_g1(blk + 4, q)
        issue_scatter(blk, p)
        if chain:
            cq = (q + 2) % 4
            wait_g1(blk + 2, cq)
            issue_g2add(blk + 2, cq)

    for q in range(4):
        issue_g1(q, q)
    for q in range(2):
        wait_g1(q, q)
        issue_g2add(q, q)
    visit(0, 0, 0, scatter_wait=False)
    visit(1, 1, 1, scatter_wait=False)
    visit(2, 2, 0)
    visit(3, 3, 1)

    @pl.loop(1, (N_BLKS - 8) // 4 + 1)
    def _steady(i):
        for b in range(4):
            visit(i * 4 + b, b, b % 2)

    for blk in range(N_BLKS - 6, N_BLKS):
        visit(blk, blk % 4, blk % 2,
              refill=blk + 4 < N_BLKS, chain=blk + 2 < N_BLKS)

    for p in range(2):
        wait_scatter(N_BLKS - 2 + p, p)

    plsc.subcore_barrier()

    @pl.when(s < NUM_SUBCORES - 1)
    def _out_main():
        rows = pl.ds(row_off, ROWS_MAIN)
        pltpu.sync_copy(aggr_sh.at[rows], out_hbm.at[c, rows])

    @pl.when(s == NUM_SUBCORES - 1)
    def _out_last():
        rows = pl.ds(row_off, ROWS_LAST)
        pltpu.sync_copy(aggr_sh.at[rows], out_hbm.at[c, rows])


@functools.lru_cache(maxsize=None)
def _make_sc_edge_pass(swap):
    return pl.kernel(
        functools.partial(_sc_edge_body, swap),
        out_type=jax.ShapeDtypeStruct(_SPLIT_SHAPE, jnp.float32),
        mesh=plsc.VectorSubcoreMesh(core_axis_name="c",
                                    subcore_axis_name="s"),
        compiler_params=pltpu.CompilerParams(use_tc_tiling_on_sc=False),
        scratch_types=(
            [pltpu.VMEM((EDGES_PER_TILE,), jnp.int32)] * 2
            + [pltpu.VMEM((E_BLK, HALF), jnp.float32)] * 6
            + [pltpu.VMEM_SHARED((N_NODE, HALF), jnp.float32)]
            + [pltpu.SemaphoreType.DMA] * 10
        ),
    )


def kernel(variables, factors, edge_index, edge_attr, batch_idx,
           Wm_vf, bm_vf, Wc_vf, bc_vf, Wm_fv, bm_fv, Wc_fv, bc_fv):
    del edge_attr, batch_idx
    eidx = edge_index.astype(jnp.int32)
    zeros = jnp.zeros((ROWS_LAST, HALF), jnp.float32)

    def half(w):
        return w[:EMBED], w[EMBED:]

    w1, w2 = half(Wm_vf)
    wc1, wc2 = half(Wc_vf)
    w3, w4 = half(Wm_fv)
    wc3, wc4 = half(Wc_fv)

    bm = bm_vf.reshape(1, EMBED)
    bc = bc_vf.reshape(1, EMBED)
    bm2 = bm_fv.reshape(1, EMBED)
    bc2 = bc_fv.reshape(1, EMBED)

    a_proj, b_proj = _tc_pre(factors, variables, w1, w2, bm)
    aggr1 = _make_sc_edge_pass(False)(b_proj, a_proj, eidx, zeros)
    new_factors, c_proj, d_proj = _tc_mid(
        factors, variables, aggr1, wc1, wc2, bc, w3, w4, bm2)

    aggr2 = _make_sc_edge_pass(True)(d_proj, c_proj, eidx, zeros)
    new_variables = _tc_post(variables, aggr2, wc3, wc4, bc2)

    return new_variables, new_factors
